# Initial kernel scaffold; baseline (speedup 1.0000x reference)
import jax
import jax.numpy as jnp
from jax.experimental import pallas as pl


def kernel(nfeat, edge_index, W1, b1, bn_gamma, bn_beta, W2, b2):
    raise NotImplementedError("write your pallas kernel here")


# speedup vs baseline: 4.2905x; 4.2905x over previous
import functools

import jax
import jax.numpy as jnp
from jax import lax
from jax.experimental import pallas as pl
from jax.experimental.pallas import tpu as pltpu
from jax.experimental.pallas import tpu_sc as plsc

NC = 2
NS = 16
NW = NC * NS
CHUNK = 128


def _round_up(x, m):
    return (x + m - 1) // m * m


@functools.lru_cache(maxsize=None)
def _make_sc_aggregate(n, d, npad, epad):
    n_chunks = epad // (NW * CHUNK)
    e_per_w = epad // NW
    rps = npad // NS
    zr = 8
    nz = rps // zr

    mesh = plsc.VectorSubcoreMesh(core_axis_name="c", subcore_axis_name="s")

    @functools.partial(
        pl.kernel,
        mesh=mesh,
        out_type=jax.ShapeDtypeStruct((NC, npad, d), jnp.float32),
        scratch_types=[
            pltpu.VMEM((CHUNK,), jnp.int32),
            pltpu.VMEM((CHUNK,), jnp.int32),
            pltpu.VMEM((CHUNK, d), jnp.float32),
            pltpu.VMEM((zr, d), jnp.float32),
            pltpu.VMEM_SHARED((npad, d), jnp.float32),
            pltpu.SemaphoreType.DMA,
        ],
    )
    def sc_agg(src_hbm, dst_hbm, feat_hbm, out_hbm,
               src_v, dst_v, rows_v, zero_v, agg_sh, sem):
        cid = lax.axis_index("c")
        sid = lax.axis_index("s")
        wid = sid * NC + cid

        zvec = jnp.zeros((16,), jnp.float32)
        for i in range(zr):
            for j in range(d // 16):
                zero_v[i, pl.ds(j * 16, 16)] = zvec

        row0 = sid * rps

        def zero_body(b, carry):
            pltpu.sync_copy(zero_v, agg_sh.at[pl.ds(row0 + b * zr, zr)])
            return carry
        lax.fori_loop(0, nz, zero_body, 0)

        plsc.subcore_barrier()

        base = wid * e_per_w

        def edge_body(c, carry):
            off = base + c * CHUNK
            pltpu.sync_copy(src_hbm.at[pl.ds(off, CHUNK)], src_v)
            pltpu.sync_copy(dst_hbm.at[pl.ds(off, CHUNK)], dst_v)
            pltpu.async_copy(feat_hbm.at[src_v], rows_v, sem).wait()
            pltpu.sync_copy(rows_v, agg_sh.at[dst_v], add=True)
            return carry
        lax.fori_loop(0, n_chunks, edge_body, 0)

        plsc.subcore_barrier()

        pltpu.sync_copy(agg_sh.at[pl.ds(row0, rps)],
                        out_hbm.at[cid, pl.ds(row0, rps)])

    return sc_agg


def _mlp_body(feat, a0, a1, w1, b1, g, be, w2, b2, out):
    h = feat[...] + a0[...] + a1[...]
    h = lax.dot_general(h, w1[...], (((1,), (1,)), ((), ())),
                        preferred_element_type=jnp.float32) + b1[...]
    mean = jnp.mean(h, axis=0, keepdims=True)
    c = h - mean
    var = jnp.mean(c * c, axis=0, keepdims=True)
    h = c * lax.rsqrt(var + 1e-5) * g[...] + be[...]
    h = jnp.maximum(h, 0.0)
    out[...] = lax.dot_general(h, w2[...], (((1,), (1,)), ((), ())),
                               preferred_element_type=jnp.float32) + b2[...]


def kernel(nfeat, edge_index, W1, b1, bn_gamma, bn_beta, W2, b2):
    n, d = nfeat.shape
    e = edge_index.shape[1]
    npad = _round_up(n + 1, NS * 8)
    epad = _round_up(e, NW * CHUNK)
    src = edge_index[0]
    dst = edge_index[1]
    if epad > e:
        src = jnp.concatenate([src, jnp.zeros((epad - e,), jnp.int32)])
        dst = jnp.concatenate([dst, jnp.full((epad - e,), n, jnp.int32)])
    aggs = _make_sc_aggregate(n, d, npad, epad)(src, dst, nfeat)
    out = pl.pallas_call(
        _mlp_body,
        out_shape=jax.ShapeDtypeStruct((n, d), jnp.float32),
    )(nfeat, aggs[0, :n], aggs[1, :n], W1, b1.reshape(1, d),
      bn_gamma.reshape(1, d), bn_beta.reshape(1, d), W2, b2.reshape(1, d))
    return out
